# Initial kernel scaffold; baseline (speedup 1.0000x reference)
import jax
import jax.numpy as jnp
from jax.experimental import pallas as pl


def kernel(q_u, q_b, q_sigma, delta_rule_proto, signature_proto, support_ema, ema_conf):
    raise NotImplementedError("write your pallas kernel here")


# speedup vs baseline: 1.6872x; 1.6872x over previous
import functools

import jax
import jax.numpy as jnp
from jax.experimental import pallas as pl
from jax.experimental.pallas import tpu as pltpu

U = 64
B = 128
SIG = 64
RULE = 64
CELLS = U * B
SUPPORT_MIN = 1e-4
PRIOR_MIN_POP = 4.0
PRIOR_SOFT_CAP = 0.75
USAGE_SCALE = 0.5
CONF_SCALE = 0.5
SIG_SCALE = 1.0
RET_TEMP = 1.0
SPARSE_BOOST = 1.0
LOG_FLOOR = float(jnp.log(jnp.float32(1e-6)))

T = 128


def _body(qu_ref, qb_ref, qs_ref, sigT_ref, drF_ref, spF_ref, sup_ref, conf_ref,
          dr_out, sig_out, conf_out, w_out, tw_out):
    f32 = jnp.float32

    sup = sup_ref[:]
    conf = conf_ref[:]
    valid = (sup > SUPPORT_MIN).astype(f32)

    occupied = jnp.sum(valid, keepdims=True)
    pop_scale = jnp.clip(occupied / PRIOR_MIN_POP, 0.0, 1.0)
    sp_raw = jnp.log1p(sup)
    sp = sp_raw / jnp.maximum(jnp.max(sp_raw, keepdims=True), 1.0)
    sp = jnp.clip(sp * pop_scale, 0.0, PRIOR_SOFT_CAP)
    cp = conf / jnp.maximum(jnp.max(conf, keepdims=True), 1e-6)
    cp = jnp.clip(cp * pop_scale, 0.0, PRIOR_SOFT_CAP)
    prior = USAGE_SCALE * sp + CONF_SCALE * cp
    eff_temp = RET_TEMP * (1.0 + SPARSE_BOOST * (1.0 - pop_scale))
    inv_temp = 1.0 / jnp.maximum(eff_temp, 1e-6)

    se = sigT_ref[:] + 1e-6
    sn = se / jnp.maximum(
        jnp.sqrt(jnp.sum(se * se, axis=0, keepdims=True)), 1e-12)

    qs = qs_ref[:]
    qsn = qs / jnp.maximum(
        jnp.sqrt(jnp.sum(qs * qs, axis=1, keepdims=True)), 1e-12)
    sig_score = 0.5 * (1.0 + jnp.dot(qsn, sn, preferred_element_type=f32))

    lqu = jnp.log(jnp.maximum(qu_ref[:], 1e-6))
    lqb = jnp.log(jnp.maximum(qb_ref[:], 1e-6))
    strips = [lqu[:, u:u + 1] + lqb for u in range(U)]
    jl = jnp.maximum(jnp.concatenate(strips, axis=1), LOG_FLOOR)

    logits = jl + prior + SIG_SCALE * sig_score
    z = jnp.where(valid > 0, logits, -1e9) * inv_temp
    m = jnp.max(z, axis=1, keepdims=True)
    e = jnp.exp(z - m)
    w_pre = e / jnp.sum(e, axis=1, keepdims=True)
    wv = w_pre * valid
    w = wv / jnp.maximum(jnp.sum(wv, axis=1, keepdims=True), 1e-6)

    w_out[:] = w
    dr_out[:] = jnp.dot(w, drF_ref[:], preferred_element_type=f32)
    sig_out[:] = jnp.dot(w, spF_ref[:], preferred_element_type=f32)

    tw = jnp.max(w, axis=1, keepdims=True)
    iota = jax.lax.broadcasted_iota(jnp.int32, (1, CELLS), 1)
    ti = jnp.min(jnp.where(w == tw, iota, CELLS), axis=1, keepdims=True)
    onehot = (iota == ti).astype(f32)
    top_conf = jnp.sum(onehot * cp, axis=1, keepdims=True)
    top_sig = jnp.sum(onehot * sig_score, axis=1, keepdims=True)
    tw_out[:] = tw
    conf_out[:] = jnp.clip(tw * top_conf * top_sig * pop_scale, 0.0, 1.0)


@jax.jit
def kernel(q_u, q_b, q_sigma, delta_rule_proto, signature_proto, support_ema, ema_conf):
    lead = q_u.shape[:-1]
    n = 1
    for d in lead:
        n *= d
    qu2 = q_u.reshape(n, U)
    qb2 = q_b.reshape(n, B)
    qs2 = q_sigma.reshape(n, SIG)
    sigT = signature_proto.reshape(CELLS, SIG).T
    drF = delta_rule_proto.reshape(CELLS, RULE)
    spF = signature_proto.reshape(CELLS, SIG)
    supF = support_ema.reshape(1, CELLS)
    confF = ema_conf.reshape(1, CELLS)

    grid = (n // T,)
    full = lambda shape: pl.BlockSpec(shape, lambda i: (0, 0))
    tok = lambda width: pl.BlockSpec((T, width), lambda i: (i, 0))

    dr, sig, mconf, w, tw = pl.pallas_call(
        _body,
        grid=grid,
        in_specs=[
            tok(U), tok(B), tok(SIG),
            full((SIG, CELLS)), full((CELLS, RULE)), full((CELLS, SIG)),
            full((1, CELLS)), full((1, CELLS)),
        ],
        out_specs=[tok(RULE), tok(SIG), tok(1), tok(CELLS), tok(1)],
        out_shape=[
            jax.ShapeDtypeStruct((n, RULE), jnp.float32),
            jax.ShapeDtypeStruct((n, SIG), jnp.float32),
            jax.ShapeDtypeStruct((n, 1), jnp.float32),
            jax.ShapeDtypeStruct((n, CELLS), jnp.float32),
            jax.ShapeDtypeStruct((n, 1), jnp.float32),
        ],
        compiler_params=pltpu.CompilerParams(
            dimension_semantics=("parallel",)),
    )(qu2, qb2, qs2, sigT, drF, spF, supF, confF)

    return (
        dr.reshape(lead + (RULE,)),
        sig.reshape(lead + (SIG,)),
        mconf.reshape(lead + (1,)),
        w.reshape(lead + (U, B)),
        tw.reshape(lead + (1,)),
    )
